# baseline (speedup 1.0000x reference)
import jax
import jax.numpy as jnp
from jax import lax
from jax.experimental import pallas as pl
from jax.experimental.pallas import tpu as pltpu
from jax.experimental.pallas import tpu_sc as plsc

NUM_EMBEDDINGS = 1000000
EMBED_DIM = 64
BATCH = 16384
K = 50

NC = 2
NS = 16
NW = NC * NS
B_PER_W = BATCH // NW
CHUNK = 16
N_CHUNKS = B_PER_W // CHUNK
N_PAIRS = N_CHUNKS // 2
ROWS = CHUNK * K
NSLC = EMBED_DIM // 16


def _bcast_lane(v, j):
    idx = jnp.full((16, 1), j, dtype=jnp.int32)
    dn = lax.GatherDimensionNumbers(
        offset_dims=(), collapsed_slice_dims=(0,), start_index_map=(0,))
    return lax.gather(v, idx, dn, (1,),
                      mode=lax.GatherScatterMode.PROMISE_IN_BOUNDS)


def _stage_meta(g, base, ids_hbm, rat_hbm, idx_v, rat_v, semi):
    row0 = base + g * CHUNK
    pltpu.async_copy(ids_hbm.at[pl.ds(row0, CHUNK)], idx_v, semi)
    pltpu.async_copy(rat_hbm.at[pl.ds(row0 * K, ROWS)],
                     rat_v.at[pl.ds(0, ROWS)], semi)


def _wait_meta(ids_hbm, rat_hbm, idx_v, rat_v, semi):
    pltpu.make_async_copy(ids_hbm.at[pl.ds(0, CHUNK)], idx_v, semi).wait()
    pltpu.make_async_copy(rat_hbm.at[pl.ds(0, ROWS)],
                          rat_v.at[pl.ds(0, ROWS)], semi).wait()


def _issue_gathers(table_hbm, idx_v, rows_v, semg):
    for b in range(CHUNK):
        pltpu.async_copy(table_hbm.at[idx_v.at[b]],
                         rows_v.at[pl.ds(b * K, K)], semg)


def _compute(g, base, table_hbm, out_hbm, rat_v, rows_v, out_v, semg):
    pltpu.make_async_copy(table_hbm.at[pl.ds(0, ROWS)], rows_v, semg).wait()

    def b_body(b, _):
        acc = [jnp.zeros((16,), jnp.float32) for _ in range(2 * NSLC)]
        for kk in range(0, K, 16):
            nj = min(16, K - kk)
            wv = rat_v[pl.ds(b * K + kk, 16)]
            wv = (wv - 3.5) * (1.0 / K)
            for j in range(nj):
                wb = _bcast_lane(wv, j)
                r = b * K + kk + j
                p = (j % 2) * NSLC
                for s in range(NSLC):
                    acc[p + s] = acc[p + s] + wb * rows_v[r, pl.ds(s * 16, 16)]
        for s in range(NSLC):
            out_v[pl.ds(b * EMBED_DIM + s * 16, 16)] = acc[s] + acc[NSLC + s]
        return ()

    lax.fori_loop(0, CHUNK, b_body, ())

    row0 = base + g * CHUNK
    pltpu.sync_copy(out_v, out_hbm.at[pl.ds(row0 * EMBED_DIM,
                                            CHUNK * EMBED_DIM)])


def _sc_body(ids_hbm, rat_hbm, table_hbm, out_hbm,
             idx0, idx1, rat0, rat1, rows0, rows1, out_v,
             semg0, semg1, semi0, semi1):
    wid = lax.axis_index("s") * NC + lax.axis_index("c")
    base = wid * B_PER_W

    _stage_meta(0, base, ids_hbm, rat_hbm, idx0, rat0, semi0)
    _wait_meta(ids_hbm, rat_hbm, idx0, rat0, semi0)
    _issue_gathers(table_hbm, idx0, rows0, semg0)
    _stage_meta(1, base, ids_hbm, rat_hbm, idx1, rat1, semi1)

    def pair_body(p, _):
        g0 = 2 * p

        _wait_meta(ids_hbm, rat_hbm, idx1, rat1, semi1)
        _issue_gathers(table_hbm, idx1, rows1, semg1)
        _compute(g0, base, table_hbm, out_hbm, rat0, rows0, out_v, semg0)

        @pl.when(p < N_PAIRS - 1)
        def _():
            _stage_meta(g0 + 2, base, ids_hbm, rat_hbm, idx0, rat0, semi0)

        _compute(g0 + 1, base, table_hbm, out_hbm, rat1, rows1, out_v, semg1)

        @pl.when(p < N_PAIRS - 1)
        def _():
            _wait_meta(ids_hbm, rat_hbm, idx0, rat0, semi0)
            _issue_gathers(table_hbm, idx0, rows0, semg0)
            _stage_meta(g0 + 3, base, ids_hbm, rat_hbm, idx1, rat1, semi1)

        return ()

    lax.fori_loop(0, N_PAIRS, pair_body, ())


@jax.jit
def _sc_encoder(ids2d, rat_flat, table):
    mesh = plsc.VectorSubcoreMesh(core_axis_name="c", subcore_axis_name="s")
    return pl.kernel(
        _sc_body,
        out_type=jax.ShapeDtypeStruct((BATCH * EMBED_DIM,), jnp.float32),
        mesh=mesh,
        compiler_params=pltpu.CompilerParams(use_tc_tiling_on_sc=False),
        scratch_types=[
            pltpu.VMEM((CHUNK, K), jnp.int32),
            pltpu.VMEM((CHUNK, K), jnp.int32),
            pltpu.VMEM((ROWS + 16,), jnp.float32),
            pltpu.VMEM((ROWS + 16,), jnp.float32),
            pltpu.VMEM((ROWS, EMBED_DIM), jnp.float32),
            pltpu.VMEM((ROWS, EMBED_DIM), jnp.float32),
            pltpu.VMEM((CHUNK * EMBED_DIM,), jnp.float32),
            pltpu.SemaphoreType.DMA,
            pltpu.SemaphoreType.DMA,
            pltpu.SemaphoreType.DMA,
            pltpu.SemaphoreType.DMA,
        ],
    )(ids2d, rat_flat, table)


def kernel(movie_ids, ratings, item_emb_weight):
    ids2d = movie_ids.astype(jnp.int32)
    rat_flat = ratings.reshape(BATCH * K)
    out = _sc_encoder(ids2d, rat_flat, item_emb_weight)
    return out.reshape(BATCH, EMBED_DIM)


# speedup vs baseline: 1.0768x; 1.0768x over previous
import jax
import jax.numpy as jnp
from jax import lax
from jax.experimental import pallas as pl
from jax.experimental.pallas import tpu as pltpu
from jax.experimental.pallas import tpu_sc as plsc

NUM_EMBEDDINGS = 1000000
EMBED_DIM = 64
BATCH = 16384
K = 50

NC = 2
NS = 16
NW = NC * NS
B_PER_W = BATCH // NW
CHUNK = 16
N_CHUNKS = B_PER_W // CHUNK
ROWS = CHUNK * K
NSLC = EMBED_DIM // 16
NMETA = 4
UNROLL = 4


def _bcast_lane(v, j):
    idx = jnp.full((16, 1), j, dtype=jnp.int32)
    dn = lax.GatherDimensionNumbers(
        offset_dims=(), collapsed_slice_dims=(0,), start_index_map=(0,))
    return lax.gather(v, idx, dn, (1,),
                      mode=lax.GatherScatterMode.PROMISE_IN_BOUNDS)


def _stage_meta(g, base, ids_hbm, rat_hbm, idx_v, rat_v, semi):
    row0 = base + g * CHUNK
    pltpu.async_copy(ids_hbm.at[pl.ds(row0, CHUNK)], idx_v, semi)
    pltpu.async_copy(rat_hbm.at[pl.ds(row0 * K, ROWS)],
                     rat_v.at[pl.ds(0, ROWS)], semi)


def _wait_meta(ids_hbm, rat_hbm, idx_v, rat_v, semi):
    pltpu.make_async_copy(ids_hbm.at[pl.ds(0, CHUNK)], idx_v, semi).wait()
    pltpu.make_async_copy(rat_hbm.at[pl.ds(0, ROWS)],
                          rat_v.at[pl.ds(0, ROWS)], semi).wait()


def _issue_gathers(table_hbm, idx_v, rows_v, semg):
    for b in range(CHUNK):
        pltpu.async_copy(table_hbm.at[idx_v.at[b]],
                         rows_v.at[pl.ds(b * K, K)], semg)


def _compute(g, base, table_hbm, out_hbm, rat_v, rows_v, out_v,
             semg, semo, first):
    pltpu.make_async_copy(table_hbm.at[pl.ds(0, ROWS)], rows_v, semg).wait()

    @pl.when(jnp.logical_not(first))
    def _():
        pltpu.make_async_copy(
            out_v, out_hbm.at[pl.ds(0, CHUNK * EMBED_DIM)], semo).wait()

    def b_body(b, _):
        acc = [jnp.zeros((16,), jnp.float32) for _ in range(2 * NSLC)]
        for kk in range(0, K, 16):
            nj = min(16, K - kk)
            wv = rat_v[pl.ds(b * K + kk, 16)]
            wv = (wv - 3.5) * (1.0 / K)
            for j in range(nj):
                wb = _bcast_lane(wv, j)
                r = b * K + kk + j
                p = (j % 2) * NSLC
                for s in range(NSLC):
                    acc[p + s] = acc[p + s] + wb * rows_v[r, pl.ds(s * 16, 16)]
        for s in range(NSLC):
            out_v[pl.ds(b * EMBED_DIM + s * 16, 16)] = acc[s] + acc[NSLC + s]
        return ()

    lax.fori_loop(0, CHUNK, b_body, ())

    row0 = base + g * CHUNK
    pltpu.async_copy(out_v, out_hbm.at[pl.ds(row0 * EMBED_DIM,
                                             CHUNK * EMBED_DIM)], semo)


def _sc_body(ids_hbm, rat_hbm, table_hbm, out_hbm,
             idx_vs, rat_vs, rows_vs, out_vs, semis, semgs, semos):
    wid = lax.axis_index("s") * NC + lax.axis_index("c")
    base = wid * B_PER_W

    for g in range(2):
        _stage_meta(g, base, ids_hbm, rat_hbm, idx_vs[g], rat_vs[g], semis[g])
    _wait_meta(ids_hbm, rat_hbm, idx_vs[0], rat_vs[0], semis[0])
    _issue_gathers(table_hbm, idx_vs[0], rows_vs[0], semgs[0])
    for g in range(2, NMETA):
        _stage_meta(g, base, ids_hbm, rat_hbm, idx_vs[g], rat_vs[g], semis[g])
    _wait_meta(ids_hbm, rat_hbm, idx_vs[1], rat_vs[1], semis[1])
    _issue_gathers(table_hbm, idx_vs[1], rows_vs[1], semgs[1])

    def quad_body(p, _):
        for i in range(UNROLL):
            mi = i % NMETA
            ri = i % 2
            g = UNROLL * p + i
            _compute(g, base, table_hbm, out_hbm, rat_vs[mi], rows_vs[ri],
                     out_vs[ri], semgs[ri], semos[ri], p * UNROLL + i < 2)

            @pl.when(g < N_CHUNKS - NMETA)
            def _():
                _stage_meta(g + NMETA, base, ids_hbm, rat_hbm,
                            idx_vs[mi], rat_vs[mi], semis[mi])

            @pl.when(g < N_CHUNKS - 2)
            def _():
                _wait_meta(ids_hbm, rat_hbm, idx_vs[(mi + 2) % NMETA],
                           rat_vs[(mi + 2) % NMETA], semis[(mi + 2) % NMETA])
                _issue_gathers(table_hbm, idx_vs[(mi + 2) % NMETA],
                               rows_vs[ri], semgs[ri])

        return ()

    lax.fori_loop(0, N_CHUNKS // UNROLL, quad_body, ())

    for ri in range(2):
        pltpu.make_async_copy(
            out_vs[ri], out_hbm.at[pl.ds(0, CHUNK * EMBED_DIM)],
            semos[ri]).wait()


def _sc_entry(ids_hbm, rat_hbm, table_hbm, out_hbm,
              idx0, idx1, idx2, idx3, rat0, rat1, rat2, rat3,
              rows0, rows1, out0, out1,
              semi0, semi1, semi2, semi3, semg0, semg1, semo0, semo1):
    _sc_body(ids_hbm, rat_hbm, table_hbm, out_hbm,
             [idx0, idx1, idx2, idx3], [rat0, rat1, rat2, rat3],
             [rows0, rows1], [out0, out1],
             [semi0, semi1, semi2, semi3], [semg0, semg1], [semo0, semo1])


@jax.jit
def _sc_encoder(ids2d, rat_flat, table):
    mesh = plsc.VectorSubcoreMesh(core_axis_name="c", subcore_axis_name="s")
    return pl.kernel(
        _sc_entry,
        out_type=jax.ShapeDtypeStruct((BATCH * EMBED_DIM,), jnp.float32),
        mesh=mesh,
        compiler_params=pltpu.CompilerParams(use_tc_tiling_on_sc=False),
        scratch_types=(
            [pltpu.VMEM((CHUNK, K), jnp.int32) for _ in range(NMETA)]
            + [pltpu.VMEM((ROWS + 16,), jnp.float32) for _ in range(NMETA)]
            + [pltpu.VMEM((ROWS, EMBED_DIM), jnp.float32) for _ in range(2)]
            + [pltpu.VMEM((CHUNK * EMBED_DIM,), jnp.float32) for _ in range(2)]
            + [pltpu.SemaphoreType.DMA for _ in range(8)]
        ),
    )(ids2d, rat_flat, table)


def kernel(movie_ids, ratings, item_emb_weight):
    ids2d = movie_ids.astype(jnp.int32)
    rat_flat = ratings.reshape(BATCH * K)
    out = _sc_encoder(ids2d, rat_flat, item_emb_weight)
    return out.reshape(BATCH, EMBED_DIM)


# speedup vs baseline: 1.0831x; 1.0059x over previous
import jax
import jax.numpy as jnp
from jax import lax
from jax.experimental import pallas as pl
from jax.experimental.pallas import tpu as pltpu
from jax.experimental.pallas import tpu_sc as plsc

NUM_EMBEDDINGS = 1000000
EMBED_DIM = 64
BATCH = 16384
K = 50

NC = 2
NS = 16
NW = NC * NS
B_PER_W = BATCH // NW
CHUNK = 16
N_CHUNKS = B_PER_W // CHUNK
ROWS = CHUNK * K
NSLC = EMBED_DIM // 16
NMETA = 4
UNROLL = 4


def _bcast_lane(v, j):
    idx = jnp.full((16, 1), j, dtype=jnp.int32)
    dn = lax.GatherDimensionNumbers(
        offset_dims=(), collapsed_slice_dims=(0,), start_index_map=(0,))
    return lax.gather(v, idx, dn, (1,),
                      mode=lax.GatherScatterMode.PROMISE_IN_BOUNDS)


def _stage_meta(g, base, ids_hbm, rat_hbm, idx_v, rat_v, semi):
    row0 = base + g * CHUNK
    pltpu.async_copy(ids_hbm.at[pl.ds(row0 * K, ROWS)], idx_v, semi)
    pltpu.async_copy(rat_hbm.at[pl.ds(row0 * K, ROWS)],
                     rat_v.at[pl.ds(0, ROWS)], semi)


def _wait_meta(ids_hbm, rat_hbm, idx_v, rat_v, semi):
    pltpu.make_async_copy(ids_hbm.at[pl.ds(0, ROWS)], idx_v, semi).wait()
    pltpu.make_async_copy(rat_hbm.at[pl.ds(0, ROWS)],
                          rat_v.at[pl.ds(0, ROWS)], semi).wait()


def _issue_gathers(table_hbm, idx_v, rows_v, semg):
    pltpu.async_copy(table_hbm.at[idx_v], rows_v, semg)


def _compute(g, base, table_hbm, out_hbm, rat_v, rows_v, out_v,
             semg, semo, first):
    pltpu.make_async_copy(table_hbm.at[pl.ds(0, ROWS)], rows_v, semg).wait()

    @pl.when(jnp.logical_not(first))
    def _():
        pltpu.make_async_copy(
            out_v, out_hbm.at[pl.ds(0, CHUNK * EMBED_DIM)], semo).wait()

    def b_body(b, _):
        acc = [jnp.zeros((16,), jnp.float32) for _ in range(2 * NSLC)]
        for kk in range(0, K, 16):
            nj = min(16, K - kk)
            wv = rat_v[pl.ds(b * K + kk, 16)]
            wv = (wv - 3.5) * (1.0 / K)
            for j in range(nj):
                wb = _bcast_lane(wv, j)
                r = b * K + kk + j
                p = (j % 2) * NSLC
                for s in range(NSLC):
                    acc[p + s] = acc[p + s] + wb * rows_v[r, pl.ds(s * 16, 16)]
        for s in range(NSLC):
            out_v[pl.ds(b * EMBED_DIM + s * 16, 16)] = acc[s] + acc[NSLC + s]
        return ()

    lax.fori_loop(0, CHUNK, b_body, ())

    row0 = base + g * CHUNK
    pltpu.async_copy(out_v, out_hbm.at[pl.ds(row0 * EMBED_DIM,
                                             CHUNK * EMBED_DIM)], semo)


def _sc_body(ids_hbm, rat_hbm, table_hbm, out_hbm,
             idx_vs, rat_vs, rows_vs, out_vs, semis, semgs, semos):
    wid = lax.axis_index("s") * NC + lax.axis_index("c")
    base = wid * B_PER_W

    for g in range(2):
        _stage_meta(g, base, ids_hbm, rat_hbm, idx_vs[g], rat_vs[g], semis[g])
    _wait_meta(ids_hbm, rat_hbm, idx_vs[0], rat_vs[0], semis[0])
    _issue_gathers(table_hbm, idx_vs[0], rows_vs[0], semgs[0])
    for g in range(2, NMETA):
        _stage_meta(g, base, ids_hbm, rat_hbm, idx_vs[g], rat_vs[g], semis[g])
    _wait_meta(ids_hbm, rat_hbm, idx_vs[1], rat_vs[1], semis[1])
    _issue_gathers(table_hbm, idx_vs[1], rows_vs[1], semgs[1])

    def quad_body(p, _):
        for i in range(UNROLL):
            mi = i % NMETA
            ri = i % 2
            g = UNROLL * p + i
            _compute(g, base, table_hbm, out_hbm, rat_vs[mi], rows_vs[ri],
                     out_vs[ri], semgs[ri], semos[ri], p * UNROLL + i < 2)

            @pl.when(g < N_CHUNKS - NMETA)
            def _():
                _stage_meta(g + NMETA, base, ids_hbm, rat_hbm,
                            idx_vs[mi], rat_vs[mi], semis[mi])

            @pl.when(g < N_CHUNKS - 2)
            def _():
                _wait_meta(ids_hbm, rat_hbm, idx_vs[(mi + 2) % NMETA],
                           rat_vs[(mi + 2) % NMETA], semis[(mi + 2) % NMETA])
                _issue_gathers(table_hbm, idx_vs[(mi + 2) % NMETA],
                               rows_vs[ri], semgs[ri])

        return ()

    lax.fori_loop(0, N_CHUNKS // UNROLL, quad_body, ())

    for ri in range(2):
        pltpu.make_async_copy(
            out_vs[ri], out_hbm.at[pl.ds(0, CHUNK * EMBED_DIM)],
            semos[ri]).wait()


def _sc_entry(ids_hbm, rat_hbm, table_hbm, out_hbm,
              idx0, idx1, idx2, idx3, rat0, rat1, rat2, rat3,
              rows0, rows1, out0, out1,
              semi0, semi1, semi2, semi3, semg0, semg1, semo0, semo1):
    _sc_body(ids_hbm, rat_hbm, table_hbm, out_hbm,
             [idx0, idx1, idx2, idx3], [rat0, rat1, rat2, rat3],
             [rows0, rows1], [out0, out1],
             [semi0, semi1, semi2, semi3], [semg0, semg1], [semo0, semo1])


@jax.jit
def _sc_encoder(ids2d, rat_flat, table):
    mesh = plsc.VectorSubcoreMesh(core_axis_name="c", subcore_axis_name="s")
    return pl.kernel(
        _sc_entry,
        out_type=jax.ShapeDtypeStruct((BATCH * EMBED_DIM,), jnp.float32),
        mesh=mesh,
        compiler_params=pltpu.CompilerParams(use_tc_tiling_on_sc=False),
        scratch_types=(
            [pltpu.VMEM((ROWS,), jnp.int32) for _ in range(NMETA)]
            + [pltpu.VMEM((ROWS + 16,), jnp.float32) for _ in range(NMETA)]
            + [pltpu.VMEM((ROWS, EMBED_DIM), jnp.float32) for _ in range(2)]
            + [pltpu.VMEM((CHUNK * EMBED_DIM,), jnp.float32) for _ in range(2)]
            + [pltpu.SemaphoreType.DMA for _ in range(8)]
        ),
    )(ids2d, rat_flat, table)


def kernel(movie_ids, ratings, item_emb_weight):
    ids2d = movie_ids.astype(jnp.int32).reshape(BATCH * K)
    rat_flat = ratings.reshape(BATCH * K)
    out = _sc_encoder(ids2d, rat_flat, item_emb_weight)
    return out.reshape(BATCH, EMBED_DIM)
